# baseline (speedup 1.0000x reference)
import functools

import jax
import jax.numpy as jnp
from jax import lax
from jax.experimental import pallas as pl
from jax.experimental.pallas import tpu as pltpu
from jax.experimental.pallas import tpu_sc as plsc

_NUM_FIELDS = 26
_EMBED_DIM = 16
_BATCH = 16384
_FIELD_SIZE = 100000
_NC, _NS, _L = 2, 16, 16
_NW = _NC * _NS
_BPW = _BATCH // _NW
_C = 128
_NCHUNK = _BPW // _C


def _fm_body(x_hbm, emb_hbm, lin_hbm, out_hbm,
             xbuf, idxbuf, rows, linbuf, outbuf, gsem, lsem):
    wid = lax.axis_index("s") * _NC + lax.axis_index("c")
    base0 = wid * _BPW
    iota0 = lax.iota(jnp.int32, _L)

    def chunk_body(ci, carry):
        base = base0 + ci * _C
        pltpu.sync_copy(x_hbm.at[pl.ds(base, _C), :], xbuf)
        for f in range(_NUM_FIELDS):
            col = jnp.full((_L,), f, jnp.int32)
            off = f * _FIELD_SIZE
            for j in range(_C // _L):
                v = plsc.load_gather(xbuf, [iota0 + j * _L, col])
                idxbuf[f, pl.ds(j * _L, _L)] = v + off
        copies = []
        for f in range(_NUM_FIELDS):
            copies.append(pltpu.async_copy(emb_hbm.at[idxbuf.at[f]], rows.at[f], gsem))
            copies.append(pltpu.async_copy(lin_hbm.at[idxbuf.at[f]], linbuf.at[f], lsem))
        for cp in copies:
            cp.wait()

        iota = lax.iota(jnp.int32, _L)
        dnums = lax.GatherDimensionNumbers(
            offset_dims=(), collapsed_slice_dims=(0,), start_index_map=(0,))

        def hsum(v):
            for k in (8, 4, 2, 1):
                perm = jnp.bitwise_xor(iota, k)
                v = v + lax.gather(
                    v, perm[:, None], dimension_numbers=dnums, slice_sizes=(1,),
                    mode=lax.GatherScatterMode.PROMISE_IN_BOUNDS)
            return v

        def group_body(g, carry2):
            gbase = g * _L
            lv = linbuf[0, pl.ds(gbase, _L)]
            for f in range(1, _NUM_FIELDS):
                lv = lv + linbuf[f, pl.ds(gbase, _L)]
            acc = jnp.zeros((_L,), jnp.float32)
            for j in range(_L):
                b = gbase + j
                v = rows[0, b]
                s = v
                q = v * v
                for f in range(1, _NUM_FIELDS):
                    v = rows[f, b]
                    s = s + v
                    q = q + v * v
                r = hsum(s * s - q)
                acc = jnp.where(iota == j, r, acc)
            outbuf[pl.ds(gbase, _L)] = 0.5 * acc + lv
            return carry2

        lax.fori_loop(0, _C // _L, group_body, 0)
        pltpu.sync_copy(outbuf, out_hbm.at[pl.ds(base, _C)])
        return carry

    lax.fori_loop(0, _NCHUNK, chunk_body, 0)


@jax.jit
def _fm(x, emb_table, lin_flat):
    run = functools.partial(
        pl.kernel,
        out_type=jax.ShapeDtypeStruct((_BATCH,), jnp.float32),
        mesh=plsc.VectorSubcoreMesh(core_axis_name="c", subcore_axis_name="s"),
        compiler_params=pltpu.CompilerParams(
            use_tc_tiling_on_sc=False, needs_layout_passes=False),
        scratch_types=[
            pltpu.VMEM((_C, _NUM_FIELDS), jnp.int32),
            pltpu.VMEM((_NUM_FIELDS, _C), jnp.int32),
            pltpu.VMEM((_NUM_FIELDS, _C, _EMBED_DIM), jnp.float32),
            pltpu.VMEM((_NUM_FIELDS, _C), jnp.float32),
            pltpu.VMEM((_C,), jnp.float32),
            pltpu.SemaphoreType.DMA,
            pltpu.SemaphoreType.DMA,
        ],
    )(_fm_body)
    return run(x, emb_table, lin_flat)


def kernel(x, emb_table, lin_weight, lin_bias):
    lin_flat = lin_weight.reshape(-1)
    out = _fm(x, emb_table, lin_flat)
    return out[:, None] + lin_bias[None, :]


# speedup vs baseline: 3.4435x; 3.4435x over previous
import functools

import jax
import jax.numpy as jnp
from jax import lax
from jax.experimental import pallas as pl
from jax.experimental.pallas import tpu as pltpu
from jax.experimental.pallas import tpu_sc as plsc

_F = 26
_D = 16
_B = 16384
_FS = 100000
_V = _F * _FS
_L = 16
_BSC = _B // 2
_BLK = 2048
_PLANE = 100224


def _chunk(f):
    lo = f * _FS
    hi = min((f + 1) * _FS, _V)
    r0 = (lo // 128) * 128
    r1 = min(((hi + 127) // 128) * 128, (_V // 128) * 128)
    tail_dst = (r1 - r0) if hi > r1 else -1
    return r0, r1 - r0, lo - r0, tail_dst


def _fm_body(xt_hbm, embT_hbm, lin_hbm, embtail_hbm, lintail_hbm, out_hbm,
             plane, s_acc, m_acc, idxbuf, outbuf, ssh):
    cid = lax.axis_index("c")
    sid = lax.axis_index("s")
    b0 = cid * _BSC

    zero = jnp.zeros((_L,), jnp.float32)

    def zero_body(j, c):
        s_acc[pl.ds(j * _L, _L)] = zero
        m_acc[pl.ds(j * _L, _L)] = zero
        return c

    lax.fori_loop(0, _BSC // _L, zero_body, 0)

    def accum_chunk(f, off, is_lin):
        def blk_body(blk, c):
            pltpu.sync_copy(xt_hbm.at[f, pl.ds(b0 + blk * _BLK, _BLK)], idxbuf)

            def j_body(j, c2):
                base = blk * _BLK + j * _L
                i16 = idxbuf[pl.ds(j * _L, _L)] + off
                v = plsc.load_gather(plane, [i16])
                sl = pl.ds(base, _L)
                if is_lin:
                    m_acc[sl] = m_acc[sl] - (v + v)
                else:
                    s_acc[sl] = s_acc[sl] + v
                    m_acc[sl] = m_acc[sl] + v * v
                return c2

            lax.fori_loop(0, _BLK // _L, j_body, 0)
            return c

        lax.fori_loop(0, _BSC // _BLK, blk_body, 0)

    for f in range(_F):
        r0, ln, off, tail_dst = _chunk(f)
        pltpu.sync_copy(embT_hbm.at[sid, pl.ds(r0, ln)], plane.at[pl.ds(0, ln)])
        if tail_dst >= 0:
            pltpu.sync_copy(embtail_hbm.at[sid], plane.at[pl.ds(tail_dst, 128)])
        accum_chunk(f, off, is_lin=False)

    for f in range(_F):
        r0, ln, off, tail_dst = _chunk(f)

        @pl.when(sid == (f % 16))
        def _do_lin(f=f, r0=r0, ln=ln, off=off, tail_dst=tail_dst):
            pltpu.sync_copy(lin_hbm.at[0, pl.ds(r0, ln)], plane.at[pl.ds(0, ln)])
            if tail_dst >= 0:
                pltpu.sync_copy(lintail_hbm, plane.at[pl.ds(tail_dst, 128)])
            accum_chunk(f, off, is_lin=True)

    sub = sid * (_BSC // 16)
    nsub = _BSC // 16

    pltpu.sync_copy(s_acc, ssh.at[pl.ds(sid * _BSC, _BSC)])
    plsc.subcore_barrier()
    for d in range(_D):
        pltpu.sync_copy(ssh.at[pl.ds(d * _BSC + sub, nsub)],
                        s_acc.at[pl.ds(d * nsub, nsub)])

    def s_body(g, c):
        acc = zero
        for d in range(_D):
            sv = s_acc[pl.ds(d * nsub + g * _L, _L)]
            acc = acc + sv * sv
        outbuf[pl.ds(g * _L, _L)] = acc
        return c

    lax.fori_loop(0, nsub // _L, s_body, 0)

    plsc.subcore_barrier()
    pltpu.sync_copy(m_acc, ssh.at[pl.ds(sid * _BSC, _BSC)])
    plsc.subcore_barrier()
    for d in range(_D):
        pltpu.sync_copy(ssh.at[pl.ds(d * _BSC + sub, nsub)],
                        s_acc.at[pl.ds(d * nsub, nsub)])

    def m_body(g, c):
        mtot = zero
        for d in range(_D):
            mtot = mtot + s_acc[pl.ds(d * nsub + g * _L, _L)]
        sl = pl.ds(g * _L, _L)
        outbuf[sl] = 0.5 * (outbuf[sl] - mtot)
        return c

    lax.fori_loop(0, nsub // _L, m_body, 0)
    pltpu.sync_copy(outbuf, out_hbm.at[pl.ds(b0 + sub, nsub)])


@jax.jit
def _fm(xt, embT, lin_flat, emb_tail, lin_tail):
    run = functools.partial(
        pl.kernel,
        out_type=jax.ShapeDtypeStruct((_B,), jnp.float32),
        mesh=plsc.VectorSubcoreMesh(core_axis_name="c", subcore_axis_name="s"),
        compiler_params=pltpu.CompilerParams(
            use_tc_tiling_on_sc=True, needs_layout_passes=False,
            disable_bounds_checks=True),
        scratch_types=[
            pltpu.VMEM((_PLANE,), jnp.float32),
            pltpu.VMEM((_BSC,), jnp.float32),
            pltpu.VMEM((_BSC,), jnp.float32),
            pltpu.VMEM((_BLK,), jnp.int32),
            pltpu.VMEM((_BSC // 16,), jnp.float32),
            pltpu.VMEM_SHARED((16 * _BSC,), jnp.float32),
        ],
    )(_fm_body)
    return run(xt, embT, lin_flat, emb_tail, lin_tail)


def kernel(x, emb_table, lin_weight, lin_bias):
    emb_tail = jnp.pad(emb_table[_V - 64:].T, ((0, 0), (0, 64)))
    lin_tail = jnp.pad(lin_weight[_V - 64:, 0], (0, 64))
    out = _fm(x.T, emb_table.T, lin_weight.T, emb_tail, lin_tail)
    return out[:, None] + lin_bias[None, :]


# speedup vs baseline: 3.9803x; 1.1559x over previous
import functools

import jax
import jax.numpy as jnp
from jax import lax
from jax.experimental import pallas as pl
from jax.experimental.pallas import tpu as pltpu
from jax.experimental.pallas import tpu_sc as plsc

_F = 26
_D = 16
_B = 16384
_FS = 100000
_V = _F * _FS
_L = 16
_BSC = _B // 2
_BLK = 4096
_PLANE = 100224


def _chunk(f):
    lo = f * _FS
    hi = min((f + 1) * _FS, _V)
    r0 = (lo // 128) * 128
    r1 = min(((hi + 127) // 128) * 128, (_V // 128) * 128)
    tail_dst = (r1 - r0) if hi > r1 else -1
    return r0, r1 - r0, lo - r0, tail_dst


def _fm_body(xt_hbm, embT_hbm, lin_hbm, embtail_hbm, lintail_hbm, out_hbm,
             plane, s_acc, m_acc, idxbuf, outbuf, ssh):
    cid = lax.axis_index("c")
    sid = lax.axis_index("s")
    b0 = cid * _BSC

    zero = jnp.zeros((_L,), jnp.float32)

    def zero_body(j, c):
        s_acc[pl.ds(j * _L, _L)] = zero
        m_acc[pl.ds(j * _L, _L)] = zero
        return c

    lax.fori_loop(0, _BSC // _L, zero_body, 0)

    def accum_chunk(f, off, is_lin):
        def blk_body(blk, c):
            pltpu.sync_copy(xt_hbm.at[f, pl.ds(b0 + blk * _BLK, _BLK)], idxbuf)

            def j_body(j, c2):
                i16 = idxbuf[pl.ds(j * _L, _L)] + off
                v = plsc.load_gather(plane, [i16])
                sl = pl.ds(blk * _BLK + j * _L, _L)
                if is_lin:
                    plsc.addupdate(m_acc.at[sl], -(v + v))
                else:
                    plsc.addupdate(s_acc.at[sl], v)
                    plsc.addupdate(m_acc.at[sl], v * v)
                return c2

            lax.fori_loop(0, _BLK // _L, j_body, 0)
            return c

        lax.fori_loop(0, _BSC // _BLK, blk_body, 0)

    for f in range(_F):
        r0, ln, off, tail_dst = _chunk(f)
        pltpu.sync_copy(embT_hbm.at[sid, pl.ds(r0, ln)], plane.at[pl.ds(0, ln)])
        if tail_dst >= 0:
            pltpu.sync_copy(embtail_hbm.at[sid], plane.at[pl.ds(tail_dst, 128)])
        accum_chunk(f, off, is_lin=False)

    for f in range(_F):
        r0, ln, off, tail_dst = _chunk(f)

        @pl.when(sid == (f % 16))
        def _do_lin(f=f, r0=r0, ln=ln, off=off, tail_dst=tail_dst):
            pltpu.sync_copy(lin_hbm.at[0, pl.ds(r0, ln)], plane.at[pl.ds(0, ln)])
            if tail_dst >= 0:
                pltpu.sync_copy(lintail_hbm, plane.at[pl.ds(tail_dst, 128)])
            accum_chunk(f, off, is_lin=True)

    sub = sid * (_BSC // 16)
    nsub = _BSC // 16

    pltpu.sync_copy(s_acc, ssh.at[pl.ds(sid * _BSC, _BSC)])
    plsc.subcore_barrier()
    for d in range(_D):
        pltpu.sync_copy(ssh.at[pl.ds(d * _BSC + sub, nsub)],
                        s_acc.at[pl.ds(d * nsub, nsub)])

    def s_body(g, c):
        acc = zero
        for d in range(_D):
            sv = s_acc[pl.ds(d * nsub + g * _L, _L)]
            acc = acc + sv * sv
        outbuf[pl.ds(g * _L, _L)] = acc
        return c

    lax.fori_loop(0, nsub // _L, s_body, 0)

    plsc.subcore_barrier()
    pltpu.sync_copy(m_acc, ssh.at[pl.ds(sid * _BSC, _BSC)])
    plsc.subcore_barrier()
    for d in range(_D):
        pltpu.sync_copy(ssh.at[pl.ds(d * _BSC + sub, nsub)],
                        s_acc.at[pl.ds(d * nsub, nsub)])

    def m_body(g, c):
        mtot = zero
        for d in range(_D):
            mtot = mtot + s_acc[pl.ds(d * nsub + g * _L, _L)]
        sl = pl.ds(g * _L, _L)
        outbuf[sl] = 0.5 * (outbuf[sl] - mtot)
        return c

    lax.fori_loop(0, nsub // _L, m_body, 0)
    pltpu.sync_copy(outbuf, out_hbm.at[pl.ds(b0 + sub, nsub)])


@jax.jit
def _fm(xt, embT, lin_flat, emb_tail, lin_tail):
    run = functools.partial(
        pl.kernel,
        out_type=jax.ShapeDtypeStruct((_B,), jnp.float32),
        mesh=plsc.VectorSubcoreMesh(core_axis_name="c", subcore_axis_name="s"),
        compiler_params=pltpu.CompilerParams(
            use_tc_tiling_on_sc=True, needs_layout_passes=False,
            disable_bounds_checks=True),
        scratch_types=[
            pltpu.VMEM((_PLANE,), jnp.float32),
            pltpu.VMEM((_BSC,), jnp.float32),
            pltpu.VMEM((_BSC,), jnp.float32),
            pltpu.VMEM((_BLK,), jnp.int32),
            pltpu.VMEM((_BSC // 16,), jnp.float32),
            pltpu.VMEM_SHARED((16 * _BSC,), jnp.float32),
        ],
    )(_fm_body)
    return run(xt, embT, lin_flat, emb_tail, lin_tail)


def kernel(x, emb_table, lin_weight, lin_bias):
    emb_tail = jnp.pad(emb_table[_V - 64:].T, ((0, 0), (0, 64)))
    lin_tail = jnp.pad(lin_weight[_V - 64:, 0], (0, 64))
    out = _fm(x.T, emb_table.T, lin_weight.T, emb_tail, lin_tail)
    return out[:, None] + lin_bias[None, :]


# speedup vs baseline: 4.1788x; 1.0499x over previous
import functools

import jax
import jax.numpy as jnp
from jax import lax
from jax.experimental import pallas as pl
from jax.experimental.pallas import tpu as pltpu
from jax.experimental.pallas import tpu_sc as plsc

_F = 26
_D = 16
_B = 16384
_FS = 100000
_V = _F * _FS
_L = 16
_BSC = _B // 2
_PLANE = 100224


def _chunk(f):
    lo = f * _FS
    hi = min((f + 1) * _FS, _V)
    r0 = (lo // 128) * 128
    r1 = min(((hi + 127) // 128) * 128, (_V // 128) * 128)
    ln = r1 - r0
    tail_dst = ln if hi > r1 else -1
    lnA = ((ln // 2) // 128) * 128
    return r0, ln, lnA, lo - r0, tail_dst


def _fm_body(xt_hbm, embT_hbm, lin_hbm, embtail_hbm, lintail_hbm, out_hbm,
             plane, s_acc, m_acc, idxbuf, outbuf, ssh, semA, semB):
    cid = lax.axis_index("c")
    sid = lax.axis_index("s")
    b0 = cid * _BSC
    zero = jnp.zeros((_L,), jnp.float32)

    def issue_a(f):
        r0, ln, lnA, off, tail = _chunk(f)
        return [pltpu.async_copy(embT_hbm.at[sid, pl.ds(r0, lnA)],
                                 plane.at[pl.ds(0, lnA)], semA)]

    def issue_b(f):
        r0, ln, lnA, off, tail = _chunk(f)
        cps = [pltpu.async_copy(embT_hbm.at[sid, pl.ds(r0 + lnA, ln - lnA)],
                                plane.at[pl.ds(lnA, ln - lnA)], semB)]
        if tail >= 0:
            cps.append(pltpu.async_copy(embtail_hbm.at[sid],
                                        plane.at[pl.ds(tail, 128)], semB))
        return cps

    cps_a = issue_a(0)
    cps_b = issue_b(0)

    def zero_body(j, c):
        s_acc[pl.ds(j * _L, _L)] = zero
        m_acc[pl.ds(j * _L, _L)] = zero
        return c

    lax.fori_loop(0, _BSC // _L, zero_body, 0)

    def gather_pass(off, lnA, half):
        def j_body(j, c):
            i16 = idxbuf[pl.ds(j * _L, _L)] + off
            msk = (i16 < lnA) if half == 0 else (i16 >= lnA)
            safe = jnp.where(msk, i16, 0)
            v = plsc.load_gather(plane, [safe])
            v = jnp.where(msk, v, 0.0)
            sl = pl.ds(j * _L, _L)
            plsc.addupdate(s_acc.at[sl], v)
            plsc.addupdate(m_acc.at[sl], v * v)
            return c

        lax.fori_loop(0, _BSC // _L, j_body, 0)

    for f in range(_F):
        r0, ln, lnA, off, tail = _chunk(f)
        pltpu.sync_copy(xt_hbm.at[f, pl.ds(b0, _BSC)], idxbuf)
        for cp in cps_a:
            cp.wait()
        gather_pass(off, lnA, 0)
        if f + 1 < _F:
            cps_a = issue_a(f + 1)
        for cp in cps_b:
            cp.wait()
        gather_pass(off, lnA, 1)
        if f + 1 < _F:
            cps_b = issue_b(f + 1)

    for f in range(_F):
        r0, ln, lnA, off, tail = _chunk(f)

        @pl.when(sid == (f % 16))
        def _do_lin(f=f, r0=r0, ln=ln, off=off, tail=tail):
            pltpu.sync_copy(lin_hbm.at[0, pl.ds(r0, ln)], plane.at[pl.ds(0, ln)])
            if tail >= 0:
                pltpu.sync_copy(lintail_hbm, plane.at[pl.ds(tail, 128)])
            pltpu.sync_copy(xt_hbm.at[f, pl.ds(b0, _BSC)], idxbuf)

            def j_body(j, c):
                i16 = idxbuf[pl.ds(j * _L, _L)] + off
                w = plsc.load_gather(plane, [i16])
                plsc.addupdate(m_acc.at[pl.ds(j * _L, _L)], -(w + w))
                return c

            lax.fori_loop(0, _BSC // _L, j_body, 0)

    sub = sid * (_BSC // 16)
    nsub = _BSC // 16
    xbase = cid * 16 * _BSC

    pltpu.sync_copy(s_acc, ssh.at[pl.ds(xbase + sid * _BSC, _BSC)])
    plsc.subcore_barrier()
    for d in range(_D):
        pltpu.sync_copy(ssh.at[pl.ds(xbase + d * _BSC + sub, nsub)],
                        s_acc.at[pl.ds(d * nsub, nsub)])

    def s_body(g, c):
        acc = zero
        for d in range(_D):
            sv = s_acc[pl.ds(d * nsub + g * _L, _L)]
            acc = acc + sv * sv
        outbuf[pl.ds(g * _L, _L)] = acc
        return c

    lax.fori_loop(0, nsub // _L, s_body, 0)

    plsc.subcore_barrier()
    pltpu.sync_copy(m_acc, ssh.at[pl.ds(xbase + sid * _BSC, _BSC)])
    plsc.subcore_barrier()
    for d in range(_D):
        pltpu.sync_copy(ssh.at[pl.ds(xbase + d * _BSC + sub, nsub)],
                        s_acc.at[pl.ds(d * nsub, nsub)])

    def m_body(g, c):
        mtot = zero
        for d in range(_D):
            mtot = mtot + s_acc[pl.ds(d * nsub + g * _L, _L)]
        sl = pl.ds(g * _L, _L)
        outbuf[sl] = 0.5 * (outbuf[sl] - mtot)
        return c

    lax.fori_loop(0, nsub // _L, m_body, 0)
    pltpu.sync_copy(outbuf, out_hbm.at[pl.ds(b0 + sub, nsub)])


@jax.jit
def _fm(xt, embT, lin_flat, emb_tail, lin_tail):
    run = functools.partial(
        pl.kernel,
        out_type=jax.ShapeDtypeStruct((_B,), jnp.float32),
        mesh=plsc.VectorSubcoreMesh(core_axis_name="c", subcore_axis_name="s"),
        compiler_params=pltpu.CompilerParams(
            use_tc_tiling_on_sc=True, needs_layout_passes=False,
            disable_bounds_checks=True),
        scratch_types=[
            pltpu.VMEM((_PLANE,), jnp.float32),
            pltpu.VMEM((_BSC,), jnp.float32),
            pltpu.VMEM((_BSC,), jnp.float32),
            pltpu.VMEM((_BSC,), jnp.int32),
            pltpu.VMEM((_BSC // 16,), jnp.float32),
            pltpu.HBM((2 * 16 * _BSC,), jnp.float32),
            pltpu.SemaphoreType.DMA,
            pltpu.SemaphoreType.DMA,
        ],
    )(_fm_body)
    return run(xt, embT, lin_flat, emb_tail, lin_tail)


def kernel(x, emb_table, lin_weight, lin_bias):
    emb_tail = jnp.pad(emb_table[_V - 64:].T, ((0, 0), (0, 64)))
    lin_tail = jnp.pad(lin_weight[_V - 64:, 0], (0, 64))
    out = _fm(x.T, emb_table.T, lin_weight.T, emb_tail, lin_tail)
    return out[:, None] + lin_bias[None, :]


# speedup vs baseline: 4.4046x; 1.0540x over previous
import functools

import jax
import jax.numpy as jnp
from jax import lax
from jax.experimental import pallas as pl
from jax.experimental.pallas import tpu as pltpu
from jax.experimental.pallas import tpu_sc as plsc

_F = 26
_D = 16
_B = 16384
_FS = 100000
_V = _F * _FS
_L = 16
_BSC = _B // 2
_PLANE = 100224


def _chunk(f):
    lo = f * _FS
    hi = min((f + 1) * _FS, _V)
    r0 = (lo // 128) * 128
    r1 = min(((hi + 127) // 128) * 128, (_V // 128) * 128)
    ln = r1 - r0
    tail_dst = ln if hi > r1 else -1
    lnA = ((ln // 2) // 128) * 128
    return r0, ln, lnA, lo - r0, tail_dst


def _fm_body(xt_hbm, embT_hbm, lin_hbm, embtail_hbm, lintail_hbm, out_hbm,
             plane, s_acc, m_acc, idxbuf, outbuf, ssh, semA, semB):
    cid = lax.axis_index("c")
    sid = lax.axis_index("s")
    b0 = cid * _BSC
    zero = jnp.zeros((_L,), jnp.float32)

    def issue_a(f):
        r0, ln, lnA, off, tail = _chunk(f)
        return [pltpu.async_copy(embT_hbm.at[sid, pl.ds(r0, lnA)],
                                 plane.at[pl.ds(0, lnA)], semA)]

    def issue_b(f):
        r0, ln, lnA, off, tail = _chunk(f)
        cps = [pltpu.async_copy(embT_hbm.at[sid, pl.ds(r0 + lnA, ln - lnA)],
                                plane.at[pl.ds(lnA, ln - lnA)], semB)]
        if tail >= 0:
            cps.append(pltpu.async_copy(embtail_hbm.at[sid],
                                        plane.at[pl.ds(tail, 128)], semB))
        return cps

    cps_a = issue_a(0)
    cps_b = issue_b(0)

    def zero_body(j, c):
        s_acc[pl.ds(j * _L, _L)] = zero
        m_acc[pl.ds(j * _L, _L)] = zero
        return c

    lax.fori_loop(0, _BSC // _L, zero_body, 0)

    def gather_pass(off, lnA, half):
        def j_body(j, c):
            for u in range(2):
                sl = pl.ds(j * 2 * _L + u * _L, _L)
                i16 = idxbuf[sl] + off
                msk = (i16 < lnA) if half == 0 else (i16 >= lnA)
                v = plsc.load_gather(plane, [i16])
                v = jnp.where(msk, v, 0.0)
                plsc.addupdate(s_acc.at[sl], v)
                plsc.addupdate(m_acc.at[sl], v * v)
            return c

        lax.fori_loop(0, _BSC // (2 * _L), j_body, 0)

    for f in range(_F):
        r0, ln, lnA, off, tail = _chunk(f)
        pltpu.sync_copy(xt_hbm.at[f, pl.ds(b0, _BSC)], idxbuf)
        for cp in cps_a:
            cp.wait()
        gather_pass(off, lnA, 0)
        if f + 1 < _F:
            cps_a = issue_a(f + 1)
        for cp in cps_b:
            cp.wait()
        gather_pass(off, lnA, 1)
        if f + 1 < _F:
            cps_b = issue_b(f + 1)

    for f in range(_F):
        r0, ln, lnA, off, tail = _chunk(f)

        @pl.when(sid == (f % 16))
        def _do_lin(f=f, r0=r0, ln=ln, off=off, tail=tail):
            pltpu.sync_copy(lin_hbm.at[0, pl.ds(r0, ln)], plane.at[pl.ds(0, ln)])
            if tail >= 0:
                pltpu.sync_copy(lintail_hbm, plane.at[pl.ds(tail, 128)])
            pltpu.sync_copy(xt_hbm.at[f, pl.ds(b0, _BSC)], idxbuf)

            def j_body(j, c):
                i16 = idxbuf[pl.ds(j * _L, _L)] + off
                w = plsc.load_gather(plane, [i16])
                plsc.addupdate(m_acc.at[pl.ds(j * _L, _L)], -(w + w))
                return c

            lax.fori_loop(0, _BSC // _L, j_body, 0)

    sub = sid * (_BSC // 16)
    nsub = _BSC // 16
    xbase = cid * 16 * _BSC

    pltpu.sync_copy(s_acc, ssh.at[pl.ds(xbase + sid * _BSC, _BSC)])
    plsc.subcore_barrier()
    for d in range(_D):
        pltpu.sync_copy(ssh.at[pl.ds(xbase + d * _BSC + sub, nsub)],
                        s_acc.at[pl.ds(d * nsub, nsub)])

    def s_body(g, c):
        acc = zero
        for d in range(_D):
            sv = s_acc[pl.ds(d * nsub + g * _L, _L)]
            acc = acc + sv * sv
        outbuf[pl.ds(g * _L, _L)] = acc
        return c

    lax.fori_loop(0, nsub // _L, s_body, 0)

    plsc.subcore_barrier()
    pltpu.sync_copy(m_acc, ssh.at[pl.ds(xbase + sid * _BSC, _BSC)])
    plsc.subcore_barrier()
    for d in range(_D):
        pltpu.sync_copy(ssh.at[pl.ds(xbase + d * _BSC + sub, nsub)],
                        s_acc.at[pl.ds(d * nsub, nsub)])

    def m_body(g, c):
        mtot = zero
        for d in range(_D):
            mtot = mtot + s_acc[pl.ds(d * nsub + g * _L, _L)]
        sl = pl.ds(g * _L, _L)
        outbuf[sl] = 0.5 * (outbuf[sl] - mtot)
        return c

    lax.fori_loop(0, nsub // _L, m_body, 0)
    pltpu.sync_copy(outbuf, out_hbm.at[pl.ds(b0 + sub, nsub)])


@jax.jit
def _fm(xt, embT, lin_flat, emb_tail, lin_tail):
    run = functools.partial(
        pl.kernel,
        out_type=jax.ShapeDtypeStruct((_B,), jnp.float32),
        mesh=plsc.VectorSubcoreMesh(core_axis_name="c", subcore_axis_name="s"),
        compiler_params=pltpu.CompilerParams(
            use_tc_tiling_on_sc=True, needs_layout_passes=False,
            disable_bounds_checks=True),
        scratch_types=[
            pltpu.VMEM((_PLANE,), jnp.float32),
            pltpu.VMEM((_BSC,), jnp.float32),
            pltpu.VMEM((_BSC,), jnp.float32),
            pltpu.VMEM((_BSC,), jnp.int32),
            pltpu.VMEM((_BSC // 16,), jnp.float32),
            pltpu.HBM((2 * 16 * _BSC,), jnp.float32),
            pltpu.SemaphoreType.DMA,
            pltpu.SemaphoreType.DMA,
        ],
    )(_fm_body)
    return run(xt, embT, lin_flat, emb_tail, lin_tail)


def kernel(x, emb_table, lin_weight, lin_bias):
    emb_tail = jnp.pad(emb_table[_V - 64:].T, ((0, 0), (0, 64)))
    lin_tail = jnp.pad(lin_weight[_V - 64:, 0], (0, 64))
    out = _fm(x.T, emb_table.T, lin_weight.T, emb_tail, lin_tail)
    return out[:, None] + lin_bias[None, :]
